# baseline (speedup 1.0000x reference)
import functools

import jax
import jax.numpy as jnp
from jax.experimental import pallas as pl
from jax.experimental.pallas import tpu as pltpu

B, S, L, LD, HID, OUT = 16, 256, 75, 3, 128, 256
EPS = 1e-5
EPAD = 128


def _gelu_exact(x):
    return 0.5 * x * (1.0 + jax.lax.erf(x * (2.0 ** -0.5)))


def _layer_norm(h, g, b):
    mu = jnp.mean(h, axis=-1, keepdims=True)
    var = jnp.mean((h - mu) ** 2, axis=-1, keepdims=True)
    return (h - mu) * jax.lax.rsqrt(var + EPS) * g + b


def _body(ei_ref, xt_ref, winT_ref, bin_ref, wg0T_ref, bg0_ref, g0_ref,
          be0_ref, wg1T_ref, bg1_ref, g1_ref, be1_ref, woutT_ref, bout_ref,
          go_ref, beo_ref, out_ref, mhat_ref, *, gblk):
    f32 = jnp.float32
    dot = functools.partial(jnp.dot, preferred_element_type=f32)

    @pl.when(pl.program_id(0) == 0)
    def _build_mhat():
        iota_n = jax.lax.broadcasted_iota(jnp.int32, (L, EPAD), 0)
        r = ei_ref[0:1, :]
        c = ei_ref[1:2, :]
        Rt = (r == iota_n).astype(f32)
        Ct = (c == iota_n).astype(f32)
        At = jax.lax.dot_general(Ct, Rt, (((1,), (1,)), ((), ())),
                                 preferred_element_type=f32)
        deg_col = jnp.sum(Ct, axis=1, keepdims=True) + 1.0
        ones_row = jnp.ones((1, EPAD), f32)
        deg_row = jax.lax.dot_general(ones_row, Ct, (((1,), (1,)), ((), ())),
                                      preferred_element_type=f32) + 1.0
        dinv_c = jax.lax.rsqrt(deg_col)
        dinv_r = jax.lax.rsqrt(deg_row)
        eye = (jax.lax.broadcasted_iota(jnp.int32, (L, L), 0)
               == jax.lax.broadcasted_iota(jnp.int32, (L, L), 1)).astype(f32)
        mhat_ref[...] = (At + eye) * dinv_c * dinv_r

    bf16 = jnp.bfloat16
    mhat = mhat_ref[...]
    mhat_b = mhat.astype(bf16)
    x3 = xt_ref[...]

    wc = dot(winT_ref[...], wg0T_ref[...])
    bc = dot(bin_ref[...], wg0T_ref[...])
    rs = jnp.sum(mhat, axis=1, keepdims=True)
    xw2 = dot(x3.reshape(L * gblk, LD), wc).reshape(L, gblk * HID)
    agg3 = jax.lax.dot_general(mhat_b, xw2.astype(bf16),
                               (((1,), (0,)), ((), ())),
                               preferred_element_type=f32).reshape(L, gblk, HID)
    h3 = agg3 + rs.reshape(L, 1, 1) * bc.reshape(1, 1, HID)
    hh = h3.reshape(L * gblk, HID) + bg0_ref[...]
    h = _gelu_exact(_layer_norm(hh, g0_ref[...], be0_ref[...]))

    xw = jax.lax.dot_general(h.astype(bf16), wg1T_ref[...].astype(bf16),
                             (((1,), (0,)), ((), ())),
                             preferred_element_type=f32)
    xw2 = xw.reshape(L, gblk * HID)
    agg2 = jax.lax.dot_general(mhat_b, xw2.astype(bf16),
                               (((1,), (0,)), ((), ())),
                               preferred_element_type=f32)
    hh = agg2.reshape(L * gblk, HID) + bg1_ref[...]
    h = _gelu_exact(_layer_norm(hh, g1_ref[...], be1_ref[...]))
    pooled = jnp.mean(h.reshape(L, gblk, HID), axis=0)
    o = dot(pooled, woutT_ref[...]) + bout_ref[...]
    out_ref[...] = _layer_norm(o, go_ref[...], beo_ref[...])


def kernel(x, edge_index, W_in, b_in, Wg0, bg0, ln0_g, ln0_b, Wg1, bg1,
           ln1_g, ln1_b, W_out, b_out, lno_g, lno_b):
    Bs, Ss, _ = x.shape
    NG = Bs * Ss
    gblk = 256
    grid = NG // gblk

    xt = jnp.transpose(x.reshape(NG, L, LD), (1, 0, 2))

    E = edge_index.shape[1]
    eip = jnp.full((8, EPAD), -1, jnp.int32)
    eip = eip.at[0, :E].set(edge_index[0].astype(jnp.int32))
    eip = eip.at[1, :E].set(edge_index[1].astype(jnp.int32))

    row2 = lambda v: v.reshape(1, -1)

    full = lambda shape: pl.BlockSpec(shape, lambda b: (0,) * len(shape))
    out2 = pl.pallas_call(
        functools.partial(_body, gblk=gblk),
        grid=(grid,),
        in_specs=[
            full((8, EPAD)),
            pl.BlockSpec((L, gblk, LD), lambda b: (0, b, 0)),
            full((LD, HID)),
            full((1, HID)),
            full((HID, HID)),
            full((1, HID)),
            full((1, HID)),
            full((1, HID)),
            full((HID, HID)),
            full((1, HID)),
            full((1, HID)),
            full((1, HID)),
            full((HID, OUT)),
            full((1, OUT)),
            full((1, OUT)),
            full((1, OUT)),
        ],
        out_specs=pl.BlockSpec((gblk, OUT), lambda b: (b, 0)),
        out_shape=jax.ShapeDtypeStruct((NG, OUT), jnp.float32),
        scratch_shapes=[pltpu.VMEM((L, L), jnp.float32)],
    )(
        eip, xt,
        W_in.T, row2(b_in),
        Wg0.T, row2(bg0), row2(ln0_g), row2(ln0_b),
        Wg1.T, row2(bg1), row2(ln1_g), row2(ln1_b),
        W_out.T, row2(b_out), row2(lno_g), row2(lno_b),
    )
    return out2.reshape(Bs, Ss, OUT)


# speedup vs baseline: 1.1303x; 1.1303x over previous
import functools

import jax
import jax.numpy as jnp
from jax.experimental import pallas as pl
from jax.experimental.pallas import tpu as pltpu

B, S, L, LD, HID, OUT = 16, 256, 75, 3, 128, 256
EPS = 1e-5
EPAD = 128


def _gelu_exact(x):
    return 0.5 * x * (1.0 + jax.lax.erf(x * (2.0 ** -0.5)))


def _layer_norm(h, g, b):
    mu = jnp.mean(h, axis=-1, keepdims=True)
    var = jnp.mean((h - mu) ** 2, axis=-1, keepdims=True)
    return (h - mu) * jax.lax.rsqrt(var + EPS) * g + b


def _body(ei_ref, xt_ref, winT_ref, bin_ref, wg0T_ref, bg0_ref, g0_ref,
          be0_ref, wg1T_ref, bg1_ref, g1_ref, be1_ref, woutT_ref, bout_ref,
          go_ref, beo_ref, out_ref, mhat_ref, *, gblk):
    f32 = jnp.float32
    dot = functools.partial(jnp.dot, preferred_element_type=f32)

    @pl.when(pl.program_id(0) == 0)
    def _build_mhat():
        iota_n = jax.lax.broadcasted_iota(jnp.int32, (L, EPAD), 0)
        r = ei_ref[0:1, :]
        c = ei_ref[1:2, :]
        Rt = (r == iota_n).astype(f32)
        Ct = (c == iota_n).astype(f32)
        At = jax.lax.dot_general(Ct, Rt, (((1,), (1,)), ((), ())),
                                 preferred_element_type=f32)
        deg_col = jnp.sum(Ct, axis=1, keepdims=True) + 1.0
        ones_row = jnp.ones((1, EPAD), f32)
        deg_row = jax.lax.dot_general(ones_row, Ct, (((1,), (1,)), ((), ())),
                                      preferred_element_type=f32) + 1.0
        dinv_c = jax.lax.rsqrt(deg_col)
        dinv_r = jax.lax.rsqrt(deg_row)
        eye = (jax.lax.broadcasted_iota(jnp.int32, (L, L), 0)
               == jax.lax.broadcasted_iota(jnp.int32, (L, L), 1)).astype(f32)
        mhat_ref[...] = (At + eye) * dinv_c * dinv_r

    bf16 = jnp.bfloat16
    mhat = mhat_ref[...]
    mhat_b = mhat.astype(bf16)
    x2 = xt_ref[...]

    wc = dot(winT_ref[...], wg0T_ref[...])
    bc = dot(bin_ref[...], wg0T_ref[...])
    rs = jnp.sum(mhat, axis=1, keepdims=True)
    ax2 = dot(mhat, x2)
    proj3 = jax.lax.dot_general(ax2.reshape(L, gblk, LD), wc,
                                (((2,), (0,)), ((), ())),
                                preferred_element_type=f32)
    h3 = proj3 + rs.reshape(L, 1, 1) * bc.reshape(1, 1, HID)
    hh = h3.reshape(L * gblk, HID) + bg0_ref[...]
    h = _gelu_exact(_layer_norm(hh, g0_ref[...], be0_ref[...]))

    xw = jax.lax.dot_general(h.astype(bf16), wg1T_ref[...].astype(bf16),
                             (((1,), (0,)), ((), ())),
                             preferred_element_type=f32)
    xw2 = xw.reshape(L, gblk * HID)
    agg2 = jax.lax.dot_general(mhat_b, xw2.astype(bf16),
                               (((1,), (0,)), ((), ())),
                               preferred_element_type=f32)
    hh = agg2.reshape(L * gblk, HID) + bg1_ref[...]
    h = _gelu_exact(_layer_norm(hh, g1_ref[...], be1_ref[...]))
    pooled = jnp.mean(h.reshape(L, gblk, HID), axis=0)
    o = dot(pooled, woutT_ref[...]) + bout_ref[...]
    out_ref[...] = _layer_norm(o, go_ref[...], beo_ref[...])


def kernel(x, edge_index, W_in, b_in, Wg0, bg0, ln0_g, ln0_b, Wg1, bg1,
           ln1_g, ln1_b, W_out, b_out, lno_g, lno_b):
    Bs, Ss, _ = x.shape
    NG = Bs * Ss
    gblk = 256
    grid = NG // gblk

    xt = jnp.transpose(x.reshape(NG, L, LD), (1, 0, 2)).reshape(L, NG * LD)

    E = edge_index.shape[1]
    eip = jnp.full((8, EPAD), -1, jnp.int32)
    eip = eip.at[0, :E].set(edge_index[0].astype(jnp.int32))
    eip = eip.at[1, :E].set(edge_index[1].astype(jnp.int32))

    row2 = lambda v: v.reshape(1, -1)

    full = lambda shape: pl.BlockSpec(shape, lambda b: (0,) * len(shape))
    out2 = pl.pallas_call(
        functools.partial(_body, gblk=gblk),
        grid=(grid,),
        in_specs=[
            full((8, EPAD)),
            pl.BlockSpec((L, gblk * LD), lambda b: (0, b)),
            full((LD, HID)),
            full((1, HID)),
            full((HID, HID)),
            full((1, HID)),
            full((1, HID)),
            full((1, HID)),
            full((HID, HID)),
            full((1, HID)),
            full((1, HID)),
            full((1, HID)),
            full((HID, OUT)),
            full((1, OUT)),
            full((1, OUT)),
            full((1, OUT)),
        ],
        out_specs=pl.BlockSpec((gblk, OUT), lambda b: (b, 0)),
        out_shape=jax.ShapeDtypeStruct((NG, OUT), jnp.float32),
        scratch_shapes=[pltpu.VMEM((L, L), jnp.float32)],
    )(
        eip, xt,
        W_in.T, row2(b_in),
        Wg0.T, row2(bg0), row2(ln0_g), row2(ln0_b),
        Wg1.T, row2(bg1), row2(ln1_g), row2(ln1_b),
        W_out.T, row2(b_out), row2(lno_g), row2(lno_b),
    )
    return out2.reshape(Bs, Ss, OUT)
